# baseline (speedup 1.0000x reference)
import functools

import numpy as np
import jax
import jax.numpy as jnp
from jax import lax
from jax.experimental import pallas as pl
from jax.experimental.pallas import tpu as pltpu
from jax.experimental.pallas import tpu_sc as plsc

_HOP_SIZE = 512
_SAMPLE_RATE = 44100
_ENC_SAMPLE_RATE = 16000
_ENC_HOP_SIZE = 320
_N_FRAMES = 14000

_NUM_WORKERS = 32
_ROWS_PER_WORKER = 440
_CHUNK = 40
_NCHUNK = _ROWS_PER_WORKER // _CHUNK
_PADDED_FRAMES = _NUM_WORKERS * _ROWS_PER_WORKER


def _frame_index(num_src_rows: int) -> np.ndarray:
    ratio = _HOP_SIZE / _SAMPLE_RATE / (_ENC_HOP_SIZE / _ENC_SAMPLE_RATE)
    idx = np.round(ratio * np.arange(_N_FRAMES)).astype(np.int64)
    idx = np.minimum(idx, num_src_rows - 1).astype(np.int32)
    pad = np.zeros(_PADDED_FRAMES - _N_FRAMES, np.int32)
    return np.concatenate([idx, pad]).reshape(_NUM_WORKERS, _NCHUNK, _CHUNK)


def _aligner_body(units_h, idx_h, out_h, idx_v, buf, sem_in):
    info = plsc.get_sparse_core_info()
    wid = lax.axis_index("s") * info.num_cores + lax.axis_index("c")
    base = wid * _ROWS_PER_WORKER
    pltpu.sync_copy(idx_h.at[wid], idx_v)
    for c in range(_NCHUNK):
        pltpu.async_copy(units_h.at[idx_v.at[c]], buf, sem_in).wait()
        pltpu.sync_copy(buf, out_h.at[pl.ds(base + c * _CHUNK, _CHUNK)])


def kernel(units, n_frames):
    del n_frames
    _, num_src_rows, feat = units.shape
    table = units.reshape(num_src_rows, feat)
    idx = jnp.asarray(_frame_index(num_src_rows))

    mesh = plsc.VectorSubcoreMesh(core_axis_name="c", subcore_axis_name="s")
    run = pl.kernel(
        _aligner_body,
        out_type=jax.ShapeDtypeStruct((_PADDED_FRAMES, feat), jnp.float32),
        mesh=mesh,
        scratch_types=[
            pltpu.VMEM((_NCHUNK, _CHUNK), jnp.int32),
            pltpu.VMEM((_CHUNK, feat), jnp.float32),
            pltpu.SemaphoreType.DMA,
        ],
    )
    out = run(table, idx)
    return out[:_N_FRAMES][None]


# speedup vs baseline: 1.0692x; 1.0692x over previous
import functools

import numpy as np
import jax
import jax.numpy as jnp
from jax import lax
from jax.experimental import pallas as pl
from jax.experimental.pallas import tpu as pltpu
from jax.experimental.pallas import tpu_sc as plsc

_HOP_SIZE = 512
_SAMPLE_RATE = 44100
_ENC_SAMPLE_RATE = 16000
_ENC_HOP_SIZE = 320
_N_FRAMES = 14000

_NUM_WORKERS = 32
_ROWS_PER_WORKER = 440
_CHUNK = 40
_NCHUNK = _ROWS_PER_WORKER // _CHUNK
_PADDED_FRAMES = _NUM_WORKERS * _ROWS_PER_WORKER


def _frame_index(num_src_rows: int) -> np.ndarray:
    ratio = _HOP_SIZE / _SAMPLE_RATE / (_ENC_HOP_SIZE / _ENC_SAMPLE_RATE)
    idx = np.round(ratio * np.arange(_N_FRAMES)).astype(np.int64)
    idx = np.minimum(idx, num_src_rows - 1).astype(np.int32)
    pad = np.zeros(_PADDED_FRAMES - _N_FRAMES, np.int32)
    return np.concatenate([idx, pad]).reshape(_NUM_WORKERS, _NCHUNK, _CHUNK)


def _aligner_body(units_h, idx_h, out_h, idx_v, buf0, buf1,
                  sem_in0, sem_in1, sem_out0, sem_out1):
    info = plsc.get_sparse_core_info()
    wid = lax.axis_index("s") * info.num_cores + lax.axis_index("c")
    base = wid * _ROWS_PER_WORKER
    bufs = (buf0, buf1)
    sin = (sem_in0, sem_in1)
    sout = (sem_out0, sem_out1)
    pltpu.sync_copy(idx_h.at[wid], idx_v)

    def gather(c):
        return pltpu.async_copy(units_h.at[idx_v.at[c]], bufs[c % 2], sin[c % 2])

    def put(c):
        return pltpu.async_copy(
            bufs[c % 2], out_h.at[pl.ds(base + c * _CHUNK, _CHUNK)], sout[c % 2])

    hin = [None] * _NCHUNK
    hout = [None] * _NCHUNK
    hin[0] = gather(0)
    for c in range(_NCHUNK):
        if c + 1 < _NCHUNK:
            if c >= 1:
                hout[c - 1].wait()
            hin[c + 1] = gather(c + 1)
        hin[c].wait()
        hout[c] = put(c)
    hout[_NCHUNK - 2].wait()
    hout[_NCHUNK - 1].wait()


def kernel(units, n_frames):
    del n_frames
    _, num_src_rows, feat = units.shape
    table = units.reshape(num_src_rows, feat)
    idx = jnp.asarray(_frame_index(num_src_rows))

    mesh = plsc.VectorSubcoreMesh(core_axis_name="c", subcore_axis_name="s")
    run = pl.kernel(
        _aligner_body,
        out_type=jax.ShapeDtypeStruct((_PADDED_FRAMES, feat), jnp.float32),
        mesh=mesh,
        scratch_types=[
            pltpu.VMEM((_NCHUNK, _CHUNK), jnp.int32),
            pltpu.VMEM((_CHUNK, feat), jnp.float32),
            pltpu.VMEM((_CHUNK, feat), jnp.float32),
            pltpu.SemaphoreType.DMA,
            pltpu.SemaphoreType.DMA,
            pltpu.SemaphoreType.DMA,
            pltpu.SemaphoreType.DMA,
        ],
    )
    out = run(table, idx)
    return out[:_N_FRAMES][None]


# speedup vs baseline: 2.6412x; 2.4703x over previous
import functools

import numpy as np
import jax
import jax.numpy as jnp
from jax import lax
from jax.experimental import pallas as pl
from jax.experimental.pallas import tpu as pltpu
from jax.experimental.pallas import tpu_sc as plsc

_HOP_SIZE = 512
_SAMPLE_RATE = 44100
_ENC_SAMPLE_RATE = 16000
_ENC_HOP_SIZE = 320
_N_FRAMES = 14000

_BF = 560
_BI = 336
_NB = _N_FRAMES // _BF


def _np_index(n_frames: int, num_src_rows: int) -> np.ndarray:
    ratio = _HOP_SIZE / _SAMPLE_RATE / (_ENC_HOP_SIZE / _ENC_SAMPLE_RATE)
    idx = np.round(ratio * np.arange(n_frames)).astype(np.int64)
    return np.minimum(idx, num_src_rows - 1).astype(np.int32)


def _tc_body(starts_s, table_any, out_ref, buf, sem):
    b = pl.program_id(0)

    def span_copy(bb, slot):
        start = pl.multiple_of(starts_s[bb], 8)
        return pltpu.make_async_copy(
            table_any.at[pl.ds(start, _BI)], buf.at[slot], sem.at[slot])

    @pl.when(b == 0)
    def _():
        span_copy(0, 0).start()

    @pl.when(b + 1 < _NB)
    def _():
        span_copy(b + 1, (b + 1) % 2).start()

    span_copy(b, b % 2).wait()

    f0 = b * _BF
    s = starts_s[b]
    f2d = f0 + lax.broadcasted_iota(jnp.int32, (_BF, _BI), 0)
    r2d = s + lax.broadcasted_iota(jnp.int32, (_BF, _BI), 1)
    lo = lax.shift_right_arithmetic(882 * r2d + 71, 9)
    hi = lax.shift_right_arithmetic(882 * r2d + 953, 9)
    p = jnp.where((f2d >= lo) & (f2d < hi), 1.0, 0.0).astype(jnp.float32)
    out_ref[...] = jnp.dot(p, buf[b % 2],
                           preferred_element_type=jnp.float32)


def _tc_expand(table):
    num_src_rows, feat = table.shape
    idx = _np_index(_N_FRAMES, num_src_rows)
    starts = (idx[np.arange(_NB) * _BF] // 8) * 8
    assert int((idx[np.arange(_NB) * _BF + _BF - 1] - starts).max()) < _BI
    assert int(starts.max()) + _BI <= num_src_rows

    return pl.pallas_call(
        _tc_body,
        grid=(_NB,),
        in_specs=[
            pl.BlockSpec(memory_space=pltpu.SMEM),
            pl.BlockSpec(memory_space=pl.ANY),
        ],
        out_specs=pl.BlockSpec((_BF, feat), lambda b: (b, 0)),
        out_shape=jax.ShapeDtypeStruct((_N_FRAMES, feat), jnp.float32),
        scratch_shapes=[
            pltpu.VMEM((2, _BI, feat), jnp.float32),
            pltpu.SemaphoreType.DMA((2,)),
        ],
    )(jnp.asarray(starts), table)


def kernel(units, n_frames):
    del n_frames
    _, num_src_rows, feat = units.shape
    table = units.reshape(num_src_rows, feat)
    return _tc_expand(table)[None]
